# Initial kernel scaffold; baseline (speedup 1.0000x reference)
import jax
import jax.numpy as jnp
from jax.experimental import pallas as pl


def kernel(spd_matrix, W):
    raise NotImplementedError("write your pallas kernel here")


# speedup vs baseline: 2.0706x; 2.0706x over previous
import jax
import jax.numpy as jnp
from jax import lax
from jax.experimental import pallas as pl
from jax.experimental.pallas import tpu as pltpu
from jax.experimental.pallas import tpu_sc as plsc

_N = 1024
_D = 32
_V = 512
_B = _N * _N
_L = 16

_C = 1024


def _gather_body(idx_hbm, table_hbm, out_hbm, idx_v, table_v, rows_v, sem,
                 *, n_workers):
    mesh_nc = lax.axis_size("c")
    wid = lax.axis_index("s") * mesh_nc + lax.axis_index("c")
    bpw = _B // n_workers
    base = wid * bpw

    pltpu.sync_copy(table_hbm, table_v)

    lane32 = lax.iota(jnp.int32, _L) * _D

    def chunk(i, carry):
        off = base + i * _C
        pltpu.sync_copy(idx_hbm.at[pl.ds(off, _C)], idx_v)

        def group(g, c2):
            idx_vec = idx_v[pl.ds(g * _L, _L)]
            src = idx_vec * _D
            dst = lane32 + g * (_L * _D)
            for d in range(_D):
                vals = plsc.load_gather(table_v, [src + d])
                plsc.store_scatter(rows_v, [dst + d], vals)
            return c2

        lax.fori_loop(0, _C // _L, group, 0)
        pltpu.sync_copy(rows_v, out_hbm.at[pl.ds(off * _D, _C * _D)])
        return carry

    lax.fori_loop(0, bpw // _C, chunk, 0)


def kernel(spd_matrix, W):
    idx = spd_matrix.reshape(_B)
    mesh = plsc.VectorSubcoreMesh(core_axis_name="c", subcore_axis_name="s")
    nw = mesh.num_cores * mesh.num_subcores
    import functools
    body = functools.partial(_gather_body, n_workers=nw)
    f = pl.kernel(
        body,
        mesh=mesh,
        compiler_params=pltpu.CompilerParams(needs_layout_passes=False),
        out_type=jax.ShapeDtypeStruct((_B * _D,), jnp.float32),
        scratch_types=[
            pltpu.VMEM((_C,), jnp.int32),
            pltpu.VMEM((_V * _D,), jnp.float32),
            pltpu.VMEM((_C * _D,), jnp.float32),
            pltpu.SemaphoreType.DMA,
        ],
    )
    out = f(idx, W.reshape(_V * _D))
    return out.reshape(_N, _N, _D)


# speedup vs baseline: 4.5205x; 2.1832x over previous
import functools

import jax
import jax.numpy as jnp
from jax import lax
from jax.experimental import pallas as pl
from jax.experimental.pallas import tpu as pltpu
from jax.experimental.pallas import tpu_sc as plsc

_N = 1024
_D = 32
_V = 512
_L = 16
_U = 16


def _gather_body(idx_hbm, table_hbm, out_hbm, idx_v, table_v, rows_v, sem,
                 *, n_workers):
    mesh_nc = lax.axis_size("c")
    wid = lax.axis_index("s") * mesh_nc + lax.axis_index("c")
    rows_per_w = _N // n_workers
    base = wid * rows_per_w

    pltpu.sync_copy(table_hbm, table_v)

    def row(i, carry):
        r = base + i
        pltpu.sync_copy(idx_hbm.at[r], idx_v)

        def group(g, c2):
            b0 = g * _U
            idx_vec = idx_v[pl.ds(b0, _L)]
            for u in range(_U):
                b = b0 + u
                s = idx_vec[u]
                for h in range(_D // _L):
                    rows_v[b, pl.ds(h * _L, _L)] = table_v[s, pl.ds(h * _L, _L)]
            return c2

        lax.fori_loop(0, _N // _U, group, 0)
        pltpu.sync_copy(rows_v, out_hbm.at[r])
        return carry

    lax.fori_loop(0, rows_per_w, row, 0)


def kernel(spd_matrix, W):
    mesh = plsc.VectorSubcoreMesh(core_axis_name="c", subcore_axis_name="s")
    nw = mesh.num_cores * mesh.num_subcores
    body = functools.partial(_gather_body, n_workers=nw)
    f = pl.kernel(
        body,
        mesh=mesh,
        compiler_params=pltpu.CompilerParams(
            needs_layout_passes=False, use_tc_tiling_on_sc=False),
        out_type=jax.ShapeDtypeStruct((_N, _N, _D), jnp.float32),
        scratch_types=[
            pltpu.VMEM((_N,), jnp.int32),
            pltpu.VMEM((_V, _D), jnp.float32),
            pltpu.VMEM((_N, _D), jnp.float32),
            pltpu.SemaphoreType.DMA,
        ],
    )
    return f(spd_matrix, W)


# speedup vs baseline: 9.6388x; 2.1322x over previous
import functools

import jax
import jax.numpy as jnp
from jax import lax
from jax.experimental import pallas as pl
from jax.experimental.pallas import tpu as pltpu
from jax.experimental.pallas import tpu_sc as plsc

_N = 1024
_D = 32
_V = 512
_L = 16


def _gather_body(idx_hbm, table_hbm, out_hbm, idx_v, table_v, rows_v, sem,
                 *, n_workers):
    mesh_nc = lax.axis_size("c")
    wid = lax.axis_index("s") * mesh_nc + lax.axis_index("c")
    rows_per_w = _N // n_workers
    base = wid * rows_per_w

    pltpu.sync_copy(table_hbm, table_v)

    def row(i, carry):
        r = base + i
        pltpu.sync_copy(idx_hbm.at[r], idx_v)

        def group(g, c2):
            jb = g // 8
            j_in = (g % 8) * _L
            idx_vec = idx_v[pl.ds(g * _L, _L)]
            for d in range(_D):
                vals = plsc.load_gather(table_v, [idx_vec + d * _V])
                rows_v[d // 8, jb, d % 8, pl.ds(j_in, _L)] = vals
            return c2

        lax.fori_loop(0, _N // _L, group, 0)
        pltpu.sync_copy(rows_v, out_hbm.at[r])
        return carry

    lax.fori_loop(0, rows_per_w, row, 0)


def kernel(spd_matrix, W):
    mesh = plsc.VectorSubcoreMesh(core_axis_name="c", subcore_axis_name="s")
    nw = mesh.num_cores * mesh.num_subcores
    body = functools.partial(_gather_body, n_workers=nw)
    f = pl.kernel(
        body,
        mesh=mesh,
        compiler_params=pltpu.CompilerParams(
            needs_layout_passes=False, use_tc_tiling_on_sc=False),
        out_type=jax.ShapeDtypeStruct((_N, _D // 8, _N // 128, 8, 128),
                                      jnp.float32),
        scratch_types=[
            pltpu.VMEM((_N,), jnp.int32),
            pltpu.VMEM((_D * _V,), jnp.float32),
            pltpu.VMEM((_D // 8, _N // 128, 8, 128), jnp.float32),
            pltpu.SemaphoreType.DMA,
        ],
    )
    out = f(spd_matrix, W.T.reshape(_D * _V))
    out = jnp.transpose(out, (0, 2, 4, 1, 3))
    return out.reshape(_N, _N, _D)


# speedup vs baseline: 11.5294x; 1.1961x over previous
import functools

import jax
import jax.numpy as jnp
from jax import lax
from jax.experimental import pallas as pl
from jax.experimental.pallas import tpu as pltpu
from jax.experimental.pallas import tpu_sc as plsc

_N = 1024
_D = 32
_V = 512
_L = 16


def _gather_body(idx_hbm, table_hbm, out_hbm, idx_v, table_v,
                 rows_a, rows_b, sem_a, sem_b, *, n_workers):
    mesh_nc = lax.axis_size("c")
    wid = lax.axis_index("s") * mesh_nc + lax.axis_index("c")
    rows_per_w = _N // n_workers
    base = wid * rows_per_w

    pltpu.sync_copy(table_hbm, table_v)
    pltpu.sync_copy(idx_hbm.at[pl.ds(base, rows_per_w)], idx_v)

    def compute_row(i, rows_v):
        def group(g, c2):
            jb = g // 8
            j_in = (g % 8) * _L
            idx_vec = idx_v[i, pl.ds(g * _L, _L)]
            for d in range(_D):
                vals = plsc.load_gather(table_v, [idx_vec + d * _V])
                rows_v[d // 8, jb, d % 8, pl.ds(j_in, _L)] = vals
            return c2

        lax.fori_loop(0, _N // _L, group, 0)

    def pair(p, carry):
        for phase, rows_v, sem in ((0, rows_a, sem_a), (1, rows_b, sem_b)):
            i = p * 2 + phase
            r = base + i

            @pl.when(p > 0)
            def _wait():
                pltpu.make_async_copy(rows_v, out_hbm.at[r - 2], sem).wait()

            compute_row(i, rows_v)
            pltpu.async_copy(rows_v, out_hbm.at[r], sem)
        return carry

    npairs = rows_per_w // 2
    lax.fori_loop(0, npairs, pair, 0)
    last = base + rows_per_w
    pltpu.make_async_copy(rows_a, out_hbm.at[last - 2], sem_a).wait()
    pltpu.make_async_copy(rows_b, out_hbm.at[last - 1], sem_b).wait()


def kernel(spd_matrix, W):
    mesh = plsc.VectorSubcoreMesh(core_axis_name="c", subcore_axis_name="s")
    nw = mesh.num_cores * mesh.num_subcores
    body = functools.partial(_gather_body, n_workers=nw)
    f = pl.kernel(
        body,
        mesh=mesh,
        compiler_params=pltpu.CompilerParams(
            needs_layout_passes=False, use_tc_tiling_on_sc=False),
        out_type=jax.ShapeDtypeStruct((_N, _D // 8, _N // 128, 8, 128),
                                      jnp.float32),
        scratch_types=[
            pltpu.VMEM((_N // nw, _N), jnp.int32),
            pltpu.VMEM((_D * _V,), jnp.float32),
            pltpu.VMEM((_D // 8, _N // 128, 8, 128), jnp.float32),
            pltpu.VMEM((_D // 8, _N // 128, 8, 128), jnp.float32),
            pltpu.SemaphoreType.DMA,
            pltpu.SemaphoreType.DMA,
        ],
    )
    out = f(spd_matrix, W.T.reshape(_D * _V))
    out = jnp.transpose(out, (0, 2, 4, 1, 3))
    return out.reshape(_N, _N, _D)


# speedup vs baseline: 38.7472x; 3.3607x over previous
import functools

import jax
import jax.numpy as jnp
from jax import lax
from jax.experimental import pallas as pl
from jax.experimental.pallas import tpu as pltpu
from jax.experimental.pallas import tpu_sc as plsc

_N = 1024
_D = 32
_V = 512
_L = 16


def _gather_body(idx_hbm, table_hbm, out_hbm, idx_v, table_v,
                 rows_a, rows_b, sem_a, sem_b, *, n_workers):
    mesh_nc = lax.axis_size("c")
    wid = lax.axis_index("s") * mesh_nc + lax.axis_index("c")
    rows_per_w = _N // n_workers
    base = wid * rows_per_w

    pltpu.sync_copy(table_hbm, table_v)
    pltpu.sync_copy(idx_hbm.at[pl.ds(base, rows_per_w)], idx_v)

    def compute_row(i, rows_v):
        @plsc.parallel_loop(0, _N // _L)
        def group(g):
            jb = g // 8
            j_in = (g % 8) * _L
            idx_vec = idx_v[i, pl.ds(g * _L, _L)]
            for d in range(_D):
                vals = plsc.load_gather(table_v, [idx_vec + d * _V])
                rows_v[d // 8, jb, d % 8, pl.ds(j_in, _L)] = vals

    def pair(p, carry):
        for phase, rows_v, sem in ((0, rows_a, sem_a), (1, rows_b, sem_b)):
            i = p * 2 + phase
            r = base + i

            @pl.when(p > 0)
            def _wait():
                pltpu.make_async_copy(rows_v, out_hbm.at[r - 2], sem).wait()

            compute_row(i, rows_v)
            pltpu.async_copy(rows_v, out_hbm.at[r], sem)
        return carry

    npairs = rows_per_w // 2
    lax.fori_loop(0, npairs, pair, 0)
    last = base + rows_per_w
    pltpu.make_async_copy(rows_a, out_hbm.at[last - 2], sem_a).wait()
    pltpu.make_async_copy(rows_b, out_hbm.at[last - 1], sem_b).wait()


def kernel(spd_matrix, W):
    mesh = plsc.VectorSubcoreMesh(core_axis_name="c", subcore_axis_name="s")
    nw = mesh.num_cores * mesh.num_subcores
    body = functools.partial(_gather_body, n_workers=nw)
    f = pl.kernel(
        body,
        mesh=mesh,
        compiler_params=pltpu.CompilerParams(
            needs_layout_passes=False, use_tc_tiling_on_sc=False),
        out_type=jax.ShapeDtypeStruct((_N, _D // 8, _N // 128, 8, 128),
                                      jnp.float32),
        scratch_types=[
            pltpu.VMEM((_N // nw, _N), jnp.int32),
            pltpu.VMEM((_D * _V,), jnp.float32),
            pltpu.VMEM((_D // 8, _N // 128, 8, 128), jnp.float32),
            pltpu.VMEM((_D // 8, _N // 128, 8, 128), jnp.float32),
            pltpu.SemaphoreType.DMA,
            pltpu.SemaphoreType.DMA,
        ],
    )
    out = f(spd_matrix, W.T.reshape(_D * _V))
    out = jnp.transpose(out, (0, 2, 4, 1, 3))
    return out.reshape(_N, _N, _D)


# speedup vs baseline: 40.1505x; 1.0362x over previous
import functools

import jax
import jax.numpy as jnp
from jax import lax
from jax.experimental import pallas as pl
from jax.experimental.pallas import tpu as pltpu
from jax.experimental.pallas import tpu_sc as plsc

_N = 1024
_D = 32
_V = 512
_L = 16


def _gather_body(idx_hbm, table_hbm, out_hbm, idx_v, table_v,
                 rows_a, rows_b, sem_a, sem_b, *, n_workers):
    mesh_nc = lax.axis_size("c")
    wid = lax.axis_index("s") * mesh_nc + lax.axis_index("c")
    rows_per_w = _N // n_workers
    base = wid * rows_per_w

    pltpu.sync_copy(table_hbm, table_v)
    pltpu.sync_copy(idx_hbm.at[pl.ds(base // 8, rows_per_w // 8)], idx_v)

    def compute_row(i, rows_v):
        rb = i // 8
        r_in = i % 8

        @plsc.parallel_loop(0, _N // _L)
        def group(g):
            jb = g // 8
            j_in = (g % 8) * _L
            idx_vec = idx_v[rb, jb, r_in, pl.ds(j_in, _L)]
            for d in range(_D):
                vals = plsc.load_gather(table_v, [idx_vec + d * _V])
                rows_v[d // 8, jb, d % 8, pl.ds(j_in, _L)] = vals

    def pair(p, carry):
        for phase, rows_v, sem in ((0, rows_a, sem_a), (1, rows_b, sem_b)):
            i = p * 2 + phase
            r = base + i

            @pl.when(p > 0)
            def _wait():
                pltpu.make_async_copy(rows_v, out_hbm.at[r - 2], sem).wait()

            compute_row(i, rows_v)
            pltpu.async_copy(rows_v, out_hbm.at[r], sem)
        return carry

    npairs = rows_per_w // 2
    lax.fori_loop(0, npairs, pair, 0)
    last = base + rows_per_w
    pltpu.make_async_copy(rows_a, out_hbm.at[last - 2], sem_a).wait()
    pltpu.make_async_copy(rows_b, out_hbm.at[last - 1], sem_b).wait()


def kernel(spd_matrix, W):
    mesh = plsc.VectorSubcoreMesh(core_axis_name="c", subcore_axis_name="s")
    nw = mesh.num_cores * mesh.num_subcores
    body = functools.partial(_gather_body, n_workers=nw)
    f = pl.kernel(
        body,
        mesh=mesh,
        compiler_params=pltpu.CompilerParams(
            needs_layout_passes=False, use_tc_tiling_on_sc=False),
        out_type=jax.ShapeDtypeStruct((_N, _D // 8, _N // 128, 8, 128),
                                      jnp.float32),
        scratch_types=[
            pltpu.VMEM((_N // nw // 8, _N // 128, 8, 128), jnp.int32),
            pltpu.VMEM((_D * _V,), jnp.float32),
            pltpu.VMEM((_D // 8, _N // 128, 8, 128), jnp.float32),
            pltpu.VMEM((_D // 8, _N // 128, 8, 128), jnp.float32),
            pltpu.SemaphoreType.DMA,
            pltpu.SemaphoreType.DMA,
        ],
    )
    spd5 = spd_matrix.reshape(_N // 8, 8, _N // 128, 128)
    spd5 = jnp.transpose(spd5, (0, 2, 1, 3))
    out = f(spd5, W.T.reshape(_D * _V))
    out = jnp.transpose(out, (0, 2, 4, 1, 3))
    return out.reshape(_N, _N, _D)


# speedup vs baseline: 40.7829x; 1.0158x over previous
import functools

import jax
import jax.numpy as jnp
from jax import lax
from jax.experimental import pallas as pl
from jax.experimental.pallas import tpu as pltpu
from jax.experimental.pallas import tpu_sc as plsc

_N = 1024
_D = 32
_V = 512
_L = 16


def _gather_body(idx_hbm, table_hbm, out_hbm, idx_v, table_v,
                 rows2, sem, *, n_workers):
    mesh_nc = lax.axis_size("c")
    wid = lax.axis_index("s") * mesh_nc + lax.axis_index("c")
    rows_per_w = _N // n_workers
    base = wid * rows_per_w

    pltpu.sync_copy(table_hbm, table_v)
    pltpu.sync_copy(idx_hbm.at[pl.ds(base // 8, rows_per_w // 8)], idx_v)

    def compute_row(i, rows_v):
        rb = i // 8
        r_in = i % 8

        @plsc.parallel_loop(0, _N // _L)
        def group(g):
            jb = g // 8
            j_in = (g % 8) * _L
            idx_vec = idx_v[rb, jb, r_in, pl.ds(j_in, _L)]
            for d in range(_D):
                vals = plsc.load_gather(table_v, [idx_vec + d * _V])
                rows_v[d // 8, jb, d % 8, pl.ds(j_in, _L)] = vals

    def row(i, carry):
        r = base + i
        buf = i % 2
        rows_v = rows2.at[buf]

        @pl.when(i > 1)
        def _wait():
            pltpu.make_async_copy(rows_v, out_hbm.at[r - 2], sem).wait()

        compute_row(i, rows_v)
        pltpu.async_copy(rows_v, out_hbm.at[r], sem)
        return carry

    lax.fori_loop(0, rows_per_w, row, 0)
    last = base + rows_per_w
    pltpu.make_async_copy(rows2.at[0], out_hbm.at[last - 2], sem).wait()
    pltpu.make_async_copy(rows2.at[1], out_hbm.at[last - 1], sem).wait()


def kernel(spd_matrix, W):
    mesh = plsc.VectorSubcoreMesh(core_axis_name="c", subcore_axis_name="s")
    nw = mesh.num_cores * mesh.num_subcores
    body = functools.partial(_gather_body, n_workers=nw)
    f = pl.kernel(
        body,
        mesh=mesh,
        compiler_params=pltpu.CompilerParams(
            needs_layout_passes=False, use_tc_tiling_on_sc=False),
        out_type=jax.ShapeDtypeStruct((_N, _D // 8, _N // 128, 8, 128),
                                      jnp.float32),
        scratch_types=[
            pltpu.VMEM((_N // nw // 8, _N // 128, 8, 128), jnp.int32),
            pltpu.VMEM((_D * _V,), jnp.float32),
            pltpu.VMEM((2, _D // 8, _N // 128, 8, 128), jnp.float32),
            pltpu.SemaphoreType.DMA,
        ],
    )
    spd5 = spd_matrix.reshape(_N // 8, 8, _N // 128, 128)
    spd5 = jnp.transpose(spd5, (0, 2, 1, 3))
    out = f(spd5, W.T.reshape(_D * _V))
    out = jnp.transpose(out, (0, 2, 4, 1, 3))
    return out.reshape(_N, _N, _D)
